# baseline (speedup 1.0000x reference)
import jax
import jax.numpy as jnp
from jax import lax
from jax.experimental import pallas as pl
from jax.experimental.pallas import tpu as pltpu
from jax.experimental.pallas import tpu_sc as plsc

HID = 128
BATCH = 4096
NC = 2
NS = 16
K = 96
EB = 512


def _mesh():
    return plsc.VectorSubcoreMesh(core_axis_name="c", subcore_axis_name="s",
                                  num_cores=NC, num_subcores=NS)


def _params():
    return pltpu.CompilerParams(needs_layout_passes=False)


def _pack_edges(e, epw):
    total = NS * epw
    src = jnp.concatenate([e[0], jnp.zeros((total - e.shape[1],), jnp.int32)])
    dst = jnp.concatenate([e[1], jnp.full((total - e.shape[1],), 2**30, jnp.int32)])
    nb = epw // EB
    s3 = src.reshape(NS, nb, EB)
    d3 = dst.reshape(NS, nb, EB)
    return jnp.stack([s3, d3], axis=2).reshape(-1)


def _make_sc_seg_sum(table_shapes, rels, program, sum_rows, cnt_rows,
                     acc_rows, acc1d_rows):
    with_counts = bool(cnt_rows)
    n_tab = len(table_shapes)
    n_rel = len(rels)
    trash = acc_rows - 8

    def body(*refs):
        tabs = refs[:n_tab]
        e_refs = refs[n_tab:n_tab + n_rel]
        pos = n_tab + n_rel
        z128 = refs[pos]; pos += 1
        souts = refs[pos:pos + len(sum_rows)]; pos += len(sum_rows)
        couts = refs[pos:pos + len(cnt_rows)]; pos += len(cnt_rows)
        if with_counts:
            (acc, acc1d, ebuf, csrcF, cdstF, cs0, cd0, rows0, cs1, cd1,
             rows1, ones1, zbuf, z1d, cbuf, semg0, semg1, sems0, sems1) = refs[pos:]
        else:
            (acc, ebuf, csrcF, cdstF, cs0, cd0, rows0, cs1, cd1,
             rows1, zbuf, semg0, semg1, sems0, sems1) = refs[pos:]
            acc1d = ones1 = z1d = cbuf = None
        slots = ((cs0, cd0, rows0, semg0, sems0), (cs1, cd1, rows1, semg1, sems1))

        c = lax.axis_index("c")
        s = lax.axis_index("s")
        wid = s * NC + c
        trash_v = jnp.zeros((16,), jnp.int32) + (trash + lax.rem(wid, 8))
        zero_v = jnp.zeros((16,), jnp.int32)

        pltpu.sync_copy(z128, zbuf)
        if with_counts:
            def init1(j, _):
                ones1[pl.ds(j * 16, 16)] = jnp.ones((16,), jnp.float32)
                return 0
            lax.fori_loop(0, K // 16, init1, 0)

            def initz(j, _):
                z1d[pl.ds(j * 16, 16)] = jnp.zeros((16,), jnp.float32)
                return 0
            lax.fori_loop(0, 512 // 16, initz, 0)

        def reset_fill(j, _):
            cdstF[pl.ds(j * 16, 16)] = trash_v
            csrcF[pl.ds(j * 16, 16)] = zero_v
            return 0

        def emit_phase(sum_slot, base, R, rel_ids, cnt_off):
            r16 = R // NS
            lo = base + c * R

            def zero_sum(j, _):
                pltpu.sync_copy(zbuf, acc.at[pl.ds(s * r16 + j * 8, 8)])
                return 0
            lax.fori_loop(0, r16 // 8, zero_sum, 0)
            plsc.subcore_barrier()

            lax.fori_loop(0, K // 16, reset_fill, 0)

            for rid in rel_ids:
                tslot, epw = rels[rid]
                tab = tabs[tslot]
                ep_h = e_refs[rid]
                nb = epw // EB

                def wait_gather(q, _tab=tab):
                    csrcS, cdstS, rowsS, semg, _ = slots[q]
                    pltpu.make_async_copy(_tab.at[csrcS], rowsS, semg).wait()

                def issue_scatter(q):
                    csrcS, cdstS, rowsS, _, sems = slots[q]
                    pltpu.async_copy(rowsS, acc.at[cdstS], sems, add=True)
                    if cnt_off is not None:
                        pltpu.async_copy(ones1, acc1d.at[cdstS], sems, add=True)

                def wait_scatter(q):
                    csrcS, cdstS, rowsS, _, sems = slots[q]
                    pltpu.make_async_copy(rowsS, acc.at[cdstS], sems).wait()
                    if cnt_off is not None:
                        pltpu.make_async_copy(ones1, acc1d.at[cdstS], sems).wait()

                def handoff(q, _tab=tab):
                    csrcS, cdstS, rowsS, semg, _ = slots[q]

                    def cp(j, _):
                        cdstS[pl.ds(j * 16, 16)] = cdstF[pl.ds(j * 16, 16)]
                        csrcS[pl.ds(j * 16, 16)] = csrcF[pl.ds(j * 16, 16)]
                        return 0
                    lax.fori_loop(0, K // 16, cp, 0)
                    pltpu.async_copy(_tab.at[csrcS], rowsS, semg)
                    lax.fori_loop(0, K // 16, reset_fill, 0)

                def do_flush(cnt, par, g0, g1, s0, s1):
                    def br(q, go, so):
                        def fn():
                            lax.cond(so == 1, lambda: wait_scatter(q), lambda: None)

                            def mature():
                                wait_gather(1 - q)
                                issue_scatter(1 - q)
                            lax.cond(go == 1, mature, lambda: None)
                            handoff(q)
                        return fn
                    lax.cond(par == 0, br(0, g1, s0), br(1, g0, s1))
                    ng0 = jnp.where(par == 0, 1, 0)
                    ng1 = jnp.where(par == 0, 0, 1)
                    ns0 = jnp.where(par == 0, 0, jnp.where(g0 == 1, 1, s0))
                    ns1 = jnp.where(par == 0, jnp.where(g1 == 1, 1, s1), 0)
                    return jnp.int32(0), 1 - par, ng0, ng1, ns0, ns1

                def stage_body(b, carry, _ep_h=ep_h, _nb=nb, _do_flush=do_flush):
                    cnt, par, g0, g1, s0, s1 = carry
                    pltpu.sync_copy(_ep_h.at[pl.ds((s * _nb + b) * 2 * EB, 2 * EB)],
                                    ebuf)

                    def scan_body(i, carry):
                        cnt, par, g0, g1, s0, s1 = carry
                        dv = ebuf[pl.ds(EB + i * 16, 16)]
                        m = (dv >= lo) & (dv < lo + R)
                        mi = jnp.where(m, 1, 0).astype(jnp.int32)
                        p = plsc.cumsum(mi)
                        nm = p[15]
                        cnt, par, g0, g1, s0, s1 = lax.cond(
                            cnt + nm > K, _do_flush,
                            lambda a, b2, c2, d2, e2, f2: (a, b2, c2, d2, e2, f2),
                            cnt, par, g0, g1, s0, s1)
                        idx = cnt + p - 1
                        plsc.store_scatter(cdstF, [idx], dv - lo, mask=m)
                        sv = ebuf[pl.ds(i * 16, 16)]
                        plsc.store_scatter(csrcF, [idx], sv, mask=m)
                        return cnt + nm, par, g0, g1, s0, s1

                    return lax.fori_loop(0, EB // 16, scan_body,
                                         (cnt, par, g0, g1, s0, s1))

                z0 = jnp.int32(0)
                cnt_f, par_f, g0_f, g1_f, s0_f, s1_f = lax.fori_loop(
                    0, nb, stage_body, (z0, z0, z0, z0, z0, z0))
                for q in (0, 1):
                    gq = g0_f if q == 0 else g1_f

                    def mature_sync(_q=q, _tab=tab):
                        csrcS, cdstS, rowsS, semg, _ = slots[_q]
                        pltpu.make_async_copy(_tab.at[csrcS], rowsS, semg).wait()
                        pltpu.sync_copy(rowsS, acc.at[cdstS], add=True)
                        if cnt_off is not None:
                            pltpu.sync_copy(ones1, acc1d.at[cdstS], add=True)
                    lax.cond(gq == 1, mature_sync, lambda: None)
                for q in (0, 1):
                    sq = s0_f if q == 0 else s1_f
                    lax.cond(sq == 1, lambda _q=q: wait_scatter(_q), lambda: None)

                def final_flush(_tab=tab):
                    pltpu.sync_copy(_tab.at[csrcF], rows0)
                    pltpu.sync_copy(rows0, acc.at[cdstF], add=True)
                    if cnt_off is not None:
                        pltpu.sync_copy(ones1, acc1d.at[cdstF], add=True)
                    lax.fori_loop(0, K // 16, reset_fill, 0)
                lax.cond(cnt_f > 0, final_flush, lambda: None)

            plsc.subcore_barrier()
            obase = base + c * R + s * r16
            pltpu.sync_copy(acc.at[pl.ds(s * r16, r16)],
                            souts[sum_slot].at[pl.ds(obase, r16)])
            plsc.subcore_barrier()

        for op in program:
            if op[0] == "phase":
                _, sum_slot, base, R, rel_ids, cnt_off = op
                emit_phase(sum_slot, base, R, rel_ids, cnt_off)
            elif op[0] == "zero1d":
                _, L = op
                t16 = L // NS
                nfull, rem = t16 // 512, t16 % 512
                if nfull:
                    def zero_cnt(j, _, _t16=t16):
                        pltpu.sync_copy(z1d, acc1d.at[pl.ds(s * _t16 + j * 512, 512)])
                        return 0
                    lax.fori_loop(0, nfull, zero_cnt, 0)
                if rem:
                    pltpu.sync_copy(z1d.at[pl.ds(0, rem)],
                                    acc1d.at[pl.ds(s * t16 + nfull * 512, rem)])
            elif op[0] == "wb1d":
                _, cnt_slot, src_off, L, dst_mul = op
                t16 = L // NS
                dbase = dst_mul + c * L + s * t16
                pltpu.sync_copy(acc1d.at[pl.ds(src_off + s * t16, t16)],
                                cbuf.at[pl.ds(0, t16)])
                pltpu.sync_copy(cbuf.at[pl.ds(0, t16)],
                                couts[cnt_slot].at[pl.ds(dbase, t16)])
                plsc.subcore_barrier()

    out_type = [jax.ShapeDtypeStruct((r, HID), jnp.float32) for r in sum_rows]
    out_type += [jax.ShapeDtypeStruct((r,), jnp.float32) for r in cnt_rows]
    scratch = [pltpu.VMEM_SHARED((acc_rows, HID), jnp.float32)]
    if with_counts:
        scratch.append(pltpu.VMEM_SHARED((acc1d_rows,), jnp.float32))
    scratch += [
        pltpu.VMEM((2 * EB,), jnp.int32),
        pltpu.VMEM((K,), jnp.int32),
        pltpu.VMEM((K,), jnp.int32),
        pltpu.VMEM((K,), jnp.int32),
        pltpu.VMEM((K,), jnp.int32),
        pltpu.VMEM((K, HID), jnp.float32),
        pltpu.VMEM((K,), jnp.int32),
        pltpu.VMEM((K,), jnp.int32),
        pltpu.VMEM((K, HID), jnp.float32),
    ]
    if with_counts:
        scratch.append(pltpu.VMEM((K,), jnp.float32))
    scratch.append(pltpu.VMEM((8, HID), jnp.float32))
    if with_counts:
        scratch.append(pltpu.VMEM((512,), jnp.float32))
        scratch.append(pltpu.VMEM((800,), jnp.float32))
    scratch += [pltpu.SemaphoreType.DMA] * 4

    return pl.kernel(body, out_type=out_type, mesh=_mesh(), scratch_types=scratch,
                     compiler_params=_params())


def _sc_batch_gather(h_d2, h_c2, drug1, drug2, cell):
    per = BATCH // (NC * NS)

    def body(hd, hc, i1, i2, ic, o1, o2, oc, idx_v, rows_v):
        c = lax.axis_index("c")
        s = lax.axis_index("s")
        wid = s * NC + c
        base = wid * per
        for (ib, tab, ob) in ((i1, hd, o1), (i2, hd, o2), (ic, hc, oc)):
            pltpu.sync_copy(ib.at[pl.ds(base, per)], idx_v)
            pltpu.sync_copy(tab.at[idx_v], rows_v)
            pltpu.sync_copy(rows_v, ob.at[pl.ds(base, per)])

    out_type = [jax.ShapeDtypeStruct((BATCH, HID), jnp.float32)] * 3
    scratch = [pltpu.VMEM((per,), jnp.int32), pltpu.VMEM((per, HID), jnp.float32)]
    return pl.kernel(body, out_type=out_type, mesh=_mesh(), scratch_types=scratch,
                     compiler_params=_params())(h_d2, h_c2, drug1, drug2, cell)


def _layer_update_body(h_ref, s_ref, cnt_ref, w_ref, o_ref):
    h = h_ref[...]
    sm = s_ref[...]
    deg = jnp.maximum(cnt_ref[...], 1.0)
    x = h + sm / deg
    o_ref[...] = jnp.maximum(jnp.dot(x, w_ref[...], preferred_element_type=jnp.float32), 0.0)


def _layer_update(h, ssum, cnt, W, block=512):
    n = h.shape[0]
    grid = (pl.cdiv(n, block),)
    return pl.pallas_call(
        _layer_update_body,
        grid=grid,
        in_specs=[
            pl.BlockSpec((block, HID), lambda i: (i, 0)),
            pl.BlockSpec((block, HID), lambda i: (i, 0)),
            pl.BlockSpec((block, 1), lambda i: (i, 0)),
            pl.BlockSpec((HID, HID), lambda i: (0, 0)),
        ],
        out_specs=pl.BlockSpec((block, HID), lambda i: (i, 0)),
        out_shape=jax.ShapeDtypeStruct((n, HID), jnp.float32),
    )(h, ssum, cnt, W)


def _mlp_body(u1_ref, u2_ref, uc_ref, w1_ref, b1_ref, w2_ref, b2_ref, w3_ref, b3_ref, o_ref):
    def l2n(x):
        nrm = jnp.sqrt(jnp.sum(x * x, axis=1, keepdims=True))
        return x / jnp.maximum(nrm, 1e-12)

    hid = jnp.concatenate([l2n(u1_ref[...]), l2n(u2_ref[...]), l2n(uc_ref[...])], axis=1)
    h = jnp.maximum(jnp.dot(hid, w1_ref[...], preferred_element_type=jnp.float32) + b1_ref[...], 0.0)
    h = jnp.maximum(jnp.dot(h, w2_ref[...], preferred_element_type=jnp.float32) + b2_ref[...], 0.0)
    o_ref[...] = jnp.dot(h, w3_ref[...], preferred_element_type=jnp.float32) + b3_ref[...]


def _mlp(u1, u2, uc, w1, b1, w2, b2, w3, b3, block=512):
    grid = (BATCH // block,)
    return pl.pallas_call(
        _mlp_body,
        grid=grid,
        in_specs=[
            pl.BlockSpec((block, HID), lambda i: (i, 0)),
            pl.BlockSpec((block, HID), lambda i: (i, 0)),
            pl.BlockSpec((block, HID), lambda i: (i, 0)),
            pl.BlockSpec(w1.shape, lambda i: (0, 0)),
            pl.BlockSpec(b1.shape, lambda i: (0,)),
            pl.BlockSpec(w2.shape, lambda i: (0, 0)),
            pl.BlockSpec(b2.shape, lambda i: (0,)),
            pl.BlockSpec(w3.shape, lambda i: (0, 0)),
            pl.BlockSpec(b3.shape, lambda i: (0,)),
        ],
        out_specs=pl.BlockSpec((block, 2), lambda i: (i, 0)),
        out_shape=jax.ShapeDtypeStruct((BATCH, 2), jnp.float32),
    )(u1, u2, uc, w1, b1, w2, b2, w3, b3)


def kernel(drug_table, protein_table, cell_table, gnn_w, w1, b1, w2, b2, w3, b3,
           x_drug, x_protein, x_cell, edge_index_dp, edge_index_pd, edge_index_pp,
           edge_index_cp, edge_index_pc, drug1, drug2, cell):
    n_d, n_p, n_c = drug_table.shape[0], protein_table.shape[0], cell_table.shape[0]
    h_d, h_p, h_c = drug_table, protein_table, cell_table

    epw_dp = epw_pd = 10240
    epw_pp = 12800
    epw_cp = epw_pc = 3584
    dp_e = _pack_edges(edge_index_dp, epw_dp)
    pd_e = _pack_edges(edge_index_pd, epw_pd)
    pp_e = _pack_edges(edge_index_pp, epw_pp)
    cp_e = _pack_edges(edge_index_cp, epw_cp)
    pc_e = _pack_edges(edge_index_pc, epw_pc)

    z128 = jnp.zeros((8, HID), jnp.float32)

    seg1 = _make_sc_seg_sum(
        table_shapes=[(n_d, HID), (n_p, HID), (n_c, HID)],
        rels=[(1, epw_pd), (0, epw_dp), (1, epw_pp), (2, epw_cp), (1, epw_pc)],
        program=[
            ("zero1d", 12544),
            ("phase", 1, 0, 12544, [1, 2, 3], 0),
            ("wb1d", 1, 0, 12544, 0),
            ("zero1d", 12544),
            ("phase", 1, 25088, 12544, [1, 2, 3], 0),
            ("wb1d", 1, 0, 12544, 25088),
            ("zero1d", 5120),
            ("phase", 0, 0, 5120, [0], 0),
            ("wb1d", 0, 0, 5120, 0),
            ("zero1d", 512),
            ("phase", 2, 0, 512, [4], 0),
            ("wb1d", 2, 0, 512, 0),
        ],
        sum_rows=[10240, 50176, 1024],
        cnt_rows=[10240, 50176, 1024],
        acc_rows=12552,
        acc1d_rows=12552,
    )
    s_d, s_p, s_c, c_d, c_p, c_c = seg1(
        h_d, h_p, h_c, pd_e, dp_e, pp_e, cp_e, pc_e, z128)
    c_d2d, c_p2d, c_c2d = c_d[:, None], c_p[:, None], c_c[:, None]

    h_d1 = _layer_update(h_d, s_d, c_d2d, gnn_w[0])
    h_p1 = _layer_update(h_p, s_p, c_p2d, gnn_w[0])
    h_c1 = _layer_update(h_c, s_c, c_c2d, gnn_w[0])

    seg2 = _make_sc_seg_sum(
        table_shapes=[(n_p, HID)],
        rels=[(0, epw_pd), (0, epw_pc)],
        program=[
            ("phase", 0, 0, 5120, [0], None),
            ("phase", 1, 0, 512, [1], None),
        ],
        sum_rows=[10240, 1024],
        cnt_rows=[],
        acc_rows=5128,
        acc1d_rows=16,
    )
    s_d2, s_c2 = seg2(h_p1, pd_e, pc_e, z128)

    h_d2 = _layer_update(h_d1, s_d2, c_d2d, gnn_w[1])
    h_c2 = _layer_update(h_c1, s_c2, c_c2d, gnn_w[1])

    u1, u2, uc = _sc_batch_gather(h_d2, h_c2, drug1, drug2, cell)
    return _mlp(u1, u2, uc, w1, b1, w2, b2, w3, b3)


# speedup vs baseline: 2.4319x; 2.4319x over previous
import jax
import jax.numpy as jnp
from jax import lax
from jax.experimental import pallas as pl
from jax.experimental.pallas import tpu as pltpu
from jax.experimental.pallas import tpu_sc as plsc

HID = 128
BATCH = 4096
NC = 2
NS = 16
K = 96
EB = 512


def _mesh():
    return plsc.VectorSubcoreMesh(core_axis_name="c", subcore_axis_name="s",
                                  num_cores=NC, num_subcores=NS)


def _params():
    return pltpu.CompilerParams(needs_layout_passes=False)


def _pack_edges(e, epw):
    total = NS * epw
    src = jnp.concatenate([e[0], jnp.zeros((total - e.shape[1],), jnp.int32)])
    dst = jnp.concatenate([e[1], jnp.full((total - e.shape[1],), 2**30, jnp.int32)])
    nb = epw // EB
    s3 = src.reshape(NS, nb, EB)
    d3 = dst.reshape(NS, nb, EB)
    return jnp.stack([s3, d3], axis=2).reshape(-1)


def _make_sc_seg_sum(table_shapes, rels, program, sum_rows, cnt_rows,
                     acc_rows, acc1d_rows):
    with_counts = bool(cnt_rows)
    n_tab = len(table_shapes)
    n_rel = len(rels)
    trash = acc_rows - 8

    def body(*refs):
        tabs = refs[:n_tab]
        e_refs = refs[n_tab:n_tab + n_rel]
        pos = n_tab + n_rel
        z128 = refs[pos]; pos += 1
        souts = refs[pos:pos + len(sum_rows)]; pos += len(sum_rows)
        couts = refs[pos:pos + len(cnt_rows)]; pos += len(cnt_rows)
        if with_counts:
            (acc, acc1d, ebuf, csrcF, cdstF, cs0, cd0, rows0, cs1, cd1,
             rows1, ones1, zbuf, z1d, cbuf, semg0, semg1, sems0, sems1) = refs[pos:]
        else:
            (acc, ebuf, csrcF, cdstF, cs0, cd0, rows0, cs1, cd1,
             rows1, zbuf, semg0, semg1, sems0, sems1) = refs[pos:]
            acc1d = ones1 = z1d = cbuf = None
        slots = ((cs0, cd0, rows0, semg0, sems0), (cs1, cd1, rows1, semg1, sems1))

        c = lax.axis_index("c")
        s = lax.axis_index("s")
        wid = s * NC + c
        trash_v = jnp.zeros((16,), jnp.int32) + (trash + lax.rem(wid, 8))
        zero_v = jnp.zeros((16,), jnp.int32) + wid

        pltpu.sync_copy(z128, zbuf)
        if with_counts:
            def init1(j, _):
                ones1[pl.ds(j * 16, 16)] = jnp.ones((16,), jnp.float32)
                return 0
            lax.fori_loop(0, K // 16, init1, 0)

            def initz(j, _):
                z1d[pl.ds(j * 16, 16)] = jnp.zeros((16,), jnp.float32)
                return 0
            lax.fori_loop(0, 512 // 16, initz, 0)

        def reset_fill(j, _):
            cdstF[pl.ds(j * 16, 16)] = trash_v
            csrcF[pl.ds(j * 16, 16)] = zero_v
            return 0

        def emit_phase(sum_slot, base, R, rel_ids, cnt_off):
            r16 = R // NS
            lo = base + c * R

            def zero_sum(j, _):
                pltpu.sync_copy(zbuf, acc.at[pl.ds(s * r16 + j * 8, 8)])
                return 0
            lax.fori_loop(0, r16 // 8, zero_sum, 0)
            plsc.subcore_barrier()

            lax.fori_loop(0, K // 16, reset_fill, 0)

            for rid in rel_ids:
                tslot, epw = rels[rid]
                tab = tabs[tslot]
                ep_h = e_refs[rid]
                nb = epw // EB

                def wait_gather(q, _tab=tab):
                    csrcS, cdstS, rowsS, semg, _ = slots[q]
                    pltpu.make_async_copy(_tab.at[csrcS], rowsS, semg).wait()

                def issue_scatter(q):
                    csrcS, cdstS, rowsS, _, sems = slots[q]
                    pltpu.async_copy(rowsS, acc.at[cdstS], sems, add=True)
                    if cnt_off is not None:
                        pltpu.async_copy(ones1, acc1d.at[cdstS], sems, add=True)

                def wait_scatter(q):
                    csrcS, cdstS, rowsS, _, sems = slots[q]
                    pltpu.make_async_copy(rowsS, acc.at[cdstS], sems).wait()
                    if cnt_off is not None:
                        pltpu.make_async_copy(ones1, acc1d.at[cdstS], sems).wait()

                def handoff(q, _tab=tab):
                    csrcS, cdstS, rowsS, semg, _ = slots[q]

                    def cp(j, _):
                        cdstS[pl.ds(j * 16, 16)] = cdstF[pl.ds(j * 16, 16)]
                        csrcS[pl.ds(j * 16, 16)] = csrcF[pl.ds(j * 16, 16)]
                        return 0
                    lax.fori_loop(0, K // 16, cp, 0)
                    pltpu.async_copy(_tab.at[csrcS], rowsS, semg)
                    lax.fori_loop(0, K // 16, reset_fill, 0)

                def do_flush(cnt, par, g0, g1, s0, s1):
                    def br(q, go, so):
                        def fn():
                            lax.cond(so == 1, lambda: wait_scatter(q), lambda: None)

                            def mature():
                                wait_gather(1 - q)
                                issue_scatter(1 - q)
                            lax.cond(go == 1, mature, lambda: None)
                            handoff(q)
                        return fn
                    lax.cond(par == 0, br(0, g1, s0), br(1, g0, s1))
                    ng0 = jnp.where(par == 0, 1, 0)
                    ng1 = jnp.where(par == 0, 0, 1)
                    ns0 = jnp.where(par == 0, 0, jnp.where(g0 == 1, 1, s0))
                    ns1 = jnp.where(par == 0, jnp.where(g1 == 1, 1, s1), 0)
                    return jnp.int32(0), 1 - par, ng0, ng1, ns0, ns1

                def stage_body(b, carry, _ep_h=ep_h, _nb=nb, _do_flush=do_flush):
                    cnt, par, g0, g1, s0, s1 = carry
                    pltpu.sync_copy(_ep_h.at[pl.ds((s * _nb + b) * 2 * EB, 2 * EB)],
                                    ebuf)

                    def scan_body(i, carry):
                        cnt, par, g0, g1, s0, s1 = carry
                        dv = ebuf[pl.ds(EB + i * 16, 16)]
                        m = (dv >= lo) & (dv < lo + R)
                        mi = jnp.where(m, 1, 0).astype(jnp.int32)
                        p = plsc.cumsum(mi)
                        nm = p[15]
                        cnt, par, g0, g1, s0, s1 = lax.cond(
                            cnt + nm > K, _do_flush,
                            lambda a, b2, c2, d2, e2, f2: (a, b2, c2, d2, e2, f2),
                            cnt, par, g0, g1, s0, s1)
                        idx = cnt + p - 1
                        plsc.store_scatter(cdstF, [idx], dv - lo, mask=m)
                        sv = ebuf[pl.ds(i * 16, 16)]
                        plsc.store_scatter(csrcF, [idx], sv, mask=m)
                        return cnt + nm, par, g0, g1, s0, s1

                    return lax.fori_loop(0, EB // 16, scan_body,
                                         (cnt, par, g0, g1, s0, s1))

                z0 = jnp.int32(0)
                cnt_f, par_f, g0_f, g1_f, s0_f, s1_f = lax.fori_loop(
                    0, nb, stage_body, (z0, z0, z0, z0, z0, z0))
                for q in (0, 1):
                    gq = g0_f if q == 0 else g1_f

                    def mature_sync(_q=q, _tab=tab):
                        csrcS, cdstS, rowsS, semg, _ = slots[_q]
                        pltpu.make_async_copy(_tab.at[csrcS], rowsS, semg).wait()
                        pltpu.sync_copy(rowsS, acc.at[cdstS], add=True)
                        if cnt_off is not None:
                            pltpu.sync_copy(ones1, acc1d.at[cdstS], add=True)
                    lax.cond(gq == 1, mature_sync, lambda: None)
                for q in (0, 1):
                    sq = s0_f if q == 0 else s1_f
                    lax.cond(sq == 1, lambda _q=q: wait_scatter(_q), lambda: None)

                def final_flush(_tab=tab):
                    pltpu.sync_copy(_tab.at[csrcF], rows0)
                    pltpu.sync_copy(rows0, acc.at[cdstF], add=True)
                    if cnt_off is not None:
                        pltpu.sync_copy(ones1, acc1d.at[cdstF], add=True)
                    lax.fori_loop(0, K // 16, reset_fill, 0)
                lax.cond(cnt_f > 0, final_flush, lambda: None)

            plsc.subcore_barrier()
            obase = base + c * R + s * r16
            pltpu.sync_copy(acc.at[pl.ds(s * r16, r16)],
                            souts[sum_slot].at[pl.ds(obase, r16)])
            plsc.subcore_barrier()

        for op in program:
            if op[0] == "phase":
                _, sum_slot, base, R, rel_ids, cnt_off = op
                emit_phase(sum_slot, base, R, rel_ids, cnt_off)
            elif op[0] == "zero1d":
                _, L = op
                t16 = L // NS
                nfull, rem = t16 // 512, t16 % 512
                if nfull:
                    def zero_cnt(j, _, _t16=t16):
                        pltpu.sync_copy(z1d, acc1d.at[pl.ds(s * _t16 + j * 512, 512)])
                        return 0
                    lax.fori_loop(0, nfull, zero_cnt, 0)
                if rem:
                    pltpu.sync_copy(z1d.at[pl.ds(0, rem)],
                                    acc1d.at[pl.ds(s * t16 + nfull * 512, rem)])
            elif op[0] == "wb1d":
                _, cnt_slot, src_off, L, dst_mul = op
                t16 = L // NS
                dbase = dst_mul + c * L + s * t16
                pltpu.sync_copy(acc1d.at[pl.ds(src_off + s * t16, t16)],
                                cbuf.at[pl.ds(0, t16)])
                pltpu.sync_copy(cbuf.at[pl.ds(0, t16)],
                                couts[cnt_slot].at[pl.ds(dbase, t16)])
                plsc.subcore_barrier()

    out_type = [jax.ShapeDtypeStruct((r, HID), jnp.float32) for r in sum_rows]
    out_type += [jax.ShapeDtypeStruct((r,), jnp.float32) for r in cnt_rows]
    scratch = [pltpu.VMEM_SHARED((acc_rows, HID), jnp.float32)]
    if with_counts:
        scratch.append(pltpu.VMEM_SHARED((acc1d_rows,), jnp.float32))
    scratch += [
        pltpu.VMEM((2 * EB,), jnp.int32),
        pltpu.VMEM((K,), jnp.int32),
        pltpu.VMEM((K,), jnp.int32),
        pltpu.VMEM((K,), jnp.int32),
        pltpu.VMEM((K,), jnp.int32),
        pltpu.VMEM((K, HID), jnp.float32),
        pltpu.VMEM((K,), jnp.int32),
        pltpu.VMEM((K,), jnp.int32),
        pltpu.VMEM((K, HID), jnp.float32),
    ]
    if with_counts:
        scratch.append(pltpu.VMEM((K,), jnp.float32))
    scratch.append(pltpu.VMEM((8, HID), jnp.float32))
    if with_counts:
        scratch.append(pltpu.VMEM((512,), jnp.float32))
        scratch.append(pltpu.VMEM((800,), jnp.float32))
    scratch += [pltpu.SemaphoreType.DMA] * 4

    return pl.kernel(body, out_type=out_type, mesh=_mesh(), scratch_types=scratch,
                     compiler_params=_params())


def _sc_batch_gather(h_d2, h_c2, drug1, drug2, cell):
    per = BATCH // (NC * NS)

    def body(hd, hc, i1, i2, ic, o1, o2, oc, idx_v, rows_v):
        c = lax.axis_index("c")
        s = lax.axis_index("s")
        wid = s * NC + c
        base = wid * per
        for (ib, tab, ob) in ((i1, hd, o1), (i2, hd, o2), (ic, hc, oc)):
            pltpu.sync_copy(ib.at[pl.ds(base, per)], idx_v)
            pltpu.sync_copy(tab.at[idx_v], rows_v)
            pltpu.sync_copy(rows_v, ob.at[pl.ds(base, per)])

    out_type = [jax.ShapeDtypeStruct((BATCH, HID), jnp.float32)] * 3
    scratch = [pltpu.VMEM((per,), jnp.int32), pltpu.VMEM((per, HID), jnp.float32)]
    return pl.kernel(body, out_type=out_type, mesh=_mesh(), scratch_types=scratch,
                     compiler_params=_params())(h_d2, h_c2, drug1, drug2, cell)


def _layer_update_body(h_ref, s_ref, cnt_ref, w_ref, o_ref):
    h = h_ref[...]
    sm = s_ref[...]
    deg = jnp.maximum(cnt_ref[...], 1.0)
    x = h + sm / deg
    o_ref[...] = jnp.maximum(jnp.dot(x, w_ref[...], preferred_element_type=jnp.float32), 0.0)


def _layer_update(h, ssum, cnt, W, block=512):
    n = h.shape[0]
    grid = (pl.cdiv(n, block),)
    return pl.pallas_call(
        _layer_update_body,
        grid=grid,
        in_specs=[
            pl.BlockSpec((block, HID), lambda i: (i, 0)),
            pl.BlockSpec((block, HID), lambda i: (i, 0)),
            pl.BlockSpec((block, 1), lambda i: (i, 0)),
            pl.BlockSpec((HID, HID), lambda i: (0, 0)),
        ],
        out_specs=pl.BlockSpec((block, HID), lambda i: (i, 0)),
        out_shape=jax.ShapeDtypeStruct((n, HID), jnp.float32),
    )(h, ssum, cnt, W)


def _mlp_body(u1_ref, u2_ref, uc_ref, w1_ref, b1_ref, w2_ref, b2_ref, w3_ref, b3_ref, o_ref):
    def l2n(x):
        nrm = jnp.sqrt(jnp.sum(x * x, axis=1, keepdims=True))
        return x / jnp.maximum(nrm, 1e-12)

    hid = jnp.concatenate([l2n(u1_ref[...]), l2n(u2_ref[...]), l2n(uc_ref[...])], axis=1)
    h = jnp.maximum(jnp.dot(hid, w1_ref[...], preferred_element_type=jnp.float32) + b1_ref[...], 0.0)
    h = jnp.maximum(jnp.dot(h, w2_ref[...], preferred_element_type=jnp.float32) + b2_ref[...], 0.0)
    o_ref[...] = jnp.dot(h, w3_ref[...], preferred_element_type=jnp.float32) + b3_ref[...]


def _mlp(u1, u2, uc, w1, b1, w2, b2, w3, b3, block=512):
    grid = (BATCH // block,)
    return pl.pallas_call(
        _mlp_body,
        grid=grid,
        in_specs=[
            pl.BlockSpec((block, HID), lambda i: (i, 0)),
            pl.BlockSpec((block, HID), lambda i: (i, 0)),
            pl.BlockSpec((block, HID), lambda i: (i, 0)),
            pl.BlockSpec(w1.shape, lambda i: (0, 0)),
            pl.BlockSpec(b1.shape, lambda i: (0,)),
            pl.BlockSpec(w2.shape, lambda i: (0, 0)),
            pl.BlockSpec(b2.shape, lambda i: (0,)),
            pl.BlockSpec(w3.shape, lambda i: (0, 0)),
            pl.BlockSpec(b3.shape, lambda i: (0,)),
        ],
        out_specs=pl.BlockSpec((block, 2), lambda i: (i, 0)),
        out_shape=jax.ShapeDtypeStruct((BATCH, 2), jnp.float32),
    )(u1, u2, uc, w1, b1, w2, b2, w3, b3)


def kernel(drug_table, protein_table, cell_table, gnn_w, w1, b1, w2, b2, w3, b3,
           x_drug, x_protein, x_cell, edge_index_dp, edge_index_pd, edge_index_pp,
           edge_index_cp, edge_index_pc, drug1, drug2, cell):
    n_d, n_p, n_c = drug_table.shape[0], protein_table.shape[0], cell_table.shape[0]
    h_d, h_p, h_c = drug_table, protein_table, cell_table

    epw_dp = epw_pd = 10240
    epw_pp = 12800
    epw_cp = epw_pc = 3584
    dp_e = _pack_edges(edge_index_dp, epw_dp)
    pd_e = _pack_edges(edge_index_pd, epw_pd)
    pp_e = _pack_edges(edge_index_pp, epw_pp)
    cp_e = _pack_edges(edge_index_cp, epw_cp)
    pc_e = _pack_edges(edge_index_pc, epw_pc)

    z128 = jnp.zeros((8, HID), jnp.float32)

    seg1 = _make_sc_seg_sum(
        table_shapes=[(n_d, HID), (n_p, HID), (n_c, HID)],
        rels=[(1, epw_pd), (0, epw_dp), (1, epw_pp), (2, epw_cp), (1, epw_pc)],
        program=[
            ("zero1d", 12544),
            ("phase", 1, 0, 12544, [1, 2, 3], 0),
            ("wb1d", 1, 0, 12544, 0),
            ("zero1d", 12544),
            ("phase", 1, 25088, 12544, [1, 2, 3], 0),
            ("wb1d", 1, 0, 12544, 25088),
            ("zero1d", 5120),
            ("phase", 0, 0, 5120, [0], 0),
            ("wb1d", 0, 0, 5120, 0),
            ("zero1d", 512),
            ("phase", 2, 0, 512, [4], 0),
            ("wb1d", 2, 0, 512, 0),
        ],
        sum_rows=[10240, 50176, 1024],
        cnt_rows=[10240, 50176, 1024],
        acc_rows=12552,
        acc1d_rows=12552,
    )
    s_d, s_p, s_c, c_d, c_p, c_c = seg1(
        h_d, h_p, h_c, pd_e, dp_e, pp_e, cp_e, pc_e, z128)
    c_d2d, c_p2d, c_c2d = c_d[:, None], c_p[:, None], c_c[:, None]

    h_d1 = _layer_update(h_d, s_d, c_d2d, gnn_w[0])
    h_p1 = _layer_update(h_p, s_p, c_p2d, gnn_w[0])
    h_c1 = _layer_update(h_c, s_c, c_c2d, gnn_w[0])

    seg2 = _make_sc_seg_sum(
        table_shapes=[(n_p, HID)],
        rels=[(0, epw_pd), (0, epw_pc)],
        program=[
            ("phase", 0, 0, 5120, [0], None),
            ("phase", 1, 0, 512, [1], None),
        ],
        sum_rows=[10240, 1024],
        cnt_rows=[],
        acc_rows=5128,
        acc1d_rows=16,
    )
    s_d2, s_c2 = seg2(h_p1, pd_e, pc_e, z128)

    h_d2 = _layer_update(h_d1, s_d2, c_d2d, gnn_w[1])
    h_c2 = _layer_update(h_c1, s_c2, c_c2d, gnn_w[1])

    u1, u2, uc = _sc_batch_gather(h_d2, h_c2, drug1, drug2, cell)
    return _mlp(u1, u2, uc, w1, b1, w2, b2, w3, b3)


# speedup vs baseline: 2.4364x; 1.0018x over previous
import jax
import jax.numpy as jnp
from jax import lax
from jax.experimental import pallas as pl
from jax.experimental.pallas import tpu as pltpu
from jax.experimental.pallas import tpu_sc as plsc

HID = 128
BATCH = 4096
NC = 2
NS = 16
K = 96
EB = 640


def _mesh():
    return plsc.VectorSubcoreMesh(core_axis_name="c", subcore_axis_name="s",
                                  num_cores=NC, num_subcores=NS)


def _params():
    return pltpu.CompilerParams(needs_layout_passes=False)


def _pack_edges(e, epw):
    total = NS * epw
    src = jnp.concatenate([e[0], jnp.zeros((total - e.shape[1],), jnp.int32)])
    dst = jnp.concatenate([e[1], jnp.full((total - e.shape[1],), 2**30, jnp.int32)])
    nb = epw // EB
    s3 = src.reshape(NS, nb, EB)
    d3 = dst.reshape(NS, nb, EB)
    return jnp.stack([s3, d3], axis=2).reshape(-1)


def _make_sc_seg_sum(table_shapes, rels, program, sum_rows, cnt_rows,
                     acc_rows, acc1d_rows):
    with_counts = bool(cnt_rows)
    n_tab = len(table_shapes)
    n_rel = len(rels)
    trash = acc_rows - 8

    def body(*refs):
        tabs = refs[:n_tab]
        e_refs = refs[n_tab:n_tab + n_rel]
        pos = n_tab + n_rel
        z128 = refs[pos]; pos += 1
        souts = refs[pos:pos + len(sum_rows)]; pos += len(sum_rows)
        couts = refs[pos:pos + len(cnt_rows)]; pos += len(cnt_rows)
        if with_counts:
            (acc, acc1d, ebuf, csrcF, cdstF, cs0, cd0, rows0, cs1, cd1,
             rows1, ones1, zbuf, z1d, cbuf, semg0, semg1, sems0, sems1) = refs[pos:]
        else:
            (acc, ebuf, csrcF, cdstF, cs0, cd0, rows0, cs1, cd1,
             rows1, zbuf, semg0, semg1, sems0, sems1) = refs[pos:]
            acc1d = ones1 = z1d = cbuf = None
        slots = ((cs0, cd0, rows0, semg0, sems0), (cs1, cd1, rows1, semg1, sems1))

        c = lax.axis_index("c")
        s = lax.axis_index("s")
        wid = s * NC + c
        trash_v = jnp.zeros((16,), jnp.int32) + (trash + lax.rem(wid, 8))
        zero_v = jnp.zeros((16,), jnp.int32) + wid

        pltpu.sync_copy(z128, zbuf)
        if with_counts:
            def init1(j, _):
                ones1[pl.ds(j * 16, 16)] = jnp.ones((16,), jnp.float32)
                return 0
            lax.fori_loop(0, K // 16, init1, 0)

            def initz(j, _):
                z1d[pl.ds(j * 16, 16)] = jnp.zeros((16,), jnp.float32)
                return 0
            lax.fori_loop(0, 512 // 16, initz, 0)

        def reset_fill(j, _):
            cdstF[pl.ds(j * 16, 16)] = trash_v
            csrcF[pl.ds(j * 16, 16)] = zero_v
            return 0

        def emit_phase(sum_slot, base, R, rel_ids, cnt_off):
            r16 = R // NS
            lo = base + c * R

            def zero_sum(j, _):
                pltpu.sync_copy(zbuf, acc.at[pl.ds(s * r16 + j * 8, 8)])
                return 0
            lax.fori_loop(0, r16 // 8, zero_sum, 0)
            plsc.subcore_barrier()

            lax.fori_loop(0, K // 16, reset_fill, 0)

            for rid in rel_ids:
                tslot, epw = rels[rid]
                tab = tabs[tslot]
                ep_h = e_refs[rid]
                nb = epw // EB

                def wait_gather(q, _tab=tab):
                    csrcS, cdstS, rowsS, semg, _ = slots[q]
                    pltpu.make_async_copy(_tab.at[csrcS], rowsS, semg).wait()

                def issue_scatter(q):
                    csrcS, cdstS, rowsS, _, sems = slots[q]
                    pltpu.async_copy(rowsS, acc.at[cdstS], sems, add=True)
                    if cnt_off is not None:
                        pltpu.async_copy(ones1, acc1d.at[cdstS], sems, add=True)

                def wait_scatter(q):
                    csrcS, cdstS, rowsS, _, sems = slots[q]
                    pltpu.make_async_copy(rowsS, acc.at[cdstS], sems).wait()
                    if cnt_off is not None:
                        pltpu.make_async_copy(ones1, acc1d.at[cdstS], sems).wait()

                def handoff(q, _tab=tab):
                    csrcS, cdstS, rowsS, semg, _ = slots[q]

                    def cp(j, _):
                        cdstS[pl.ds(j * 16, 16)] = cdstF[pl.ds(j * 16, 16)]
                        csrcS[pl.ds(j * 16, 16)] = csrcF[pl.ds(j * 16, 16)]
                        return 0
                    lax.fori_loop(0, K // 16, cp, 0)
                    pltpu.async_copy(_tab.at[csrcS], rowsS, semg)
                    lax.fori_loop(0, K // 16, reset_fill, 0)

                def do_flush(cnt, par, g0, g1, s0, s1):
                    def br(q, go, so):
                        def fn():
                            lax.cond(so == 1, lambda: wait_scatter(q), lambda: None)

                            def mature():
                                wait_gather(1 - q)
                                issue_scatter(1 - q)
                            lax.cond(go == 1, mature, lambda: None)
                            handoff(q)
                        return fn
                    lax.cond(par == 0, br(0, g1, s0), br(1, g0, s1))
                    ng0 = jnp.where(par == 0, 1, 0)
                    ng1 = jnp.where(par == 0, 0, 1)
                    ns0 = jnp.where(par == 0, 0, jnp.where(g0 == 1, 1, s0))
                    ns1 = jnp.where(par == 0, jnp.where(g1 == 1, 1, s1), 0)
                    return jnp.int32(0), 1 - par, ng0, ng1, ns0, ns1

                def stage_body(b, carry, _ep_h=ep_h, _nb=nb, _do_flush=do_flush):
                    cnt, par, g0, g1, s0, s1 = carry
                    pltpu.sync_copy(_ep_h.at[pl.ds((s * _nb + b) * 2 * EB, 2 * EB)],
                                    ebuf)

                    def scan_body(i, carry):
                        cnt, par, g0, g1, s0, s1 = carry
                        dv = ebuf[pl.ds(EB + i * 16, 16)]
                        m = (dv >= lo) & (dv < lo + R)
                        mi = jnp.where(m, 1, 0).astype(jnp.int32)
                        p = plsc.cumsum(mi)
                        nm = p[15]
                        cnt, par, g0, g1, s0, s1 = lax.cond(
                            cnt + nm > K, _do_flush,
                            lambda a, b2, c2, d2, e2, f2: (a, b2, c2, d2, e2, f2),
                            cnt, par, g0, g1, s0, s1)
                        idx = cnt + p - 1
                        plsc.store_scatter(cdstF, [idx], dv - lo, mask=m)
                        sv = ebuf[pl.ds(i * 16, 16)]
                        plsc.store_scatter(csrcF, [idx], sv, mask=m)
                        return cnt + nm, par, g0, g1, s0, s1

                    return lax.fori_loop(0, EB // 16, scan_body,
                                         (cnt, par, g0, g1, s0, s1))

                z0 = jnp.int32(0)
                cnt_f, par_f, g0_f, g1_f, s0_f, s1_f = lax.fori_loop(
                    0, nb, stage_body, (z0, z0, z0, z0, z0, z0))
                for q in (0, 1):
                    gq = g0_f if q == 0 else g1_f

                    def mature_sync(_q=q, _tab=tab):
                        csrcS, cdstS, rowsS, semg, _ = slots[_q]
                        pltpu.make_async_copy(_tab.at[csrcS], rowsS, semg).wait()
                        pltpu.sync_copy(rowsS, acc.at[cdstS], add=True)
                        if cnt_off is not None:
                            pltpu.sync_copy(ones1, acc1d.at[cdstS], add=True)
                    lax.cond(gq == 1, mature_sync, lambda: None)
                for q in (0, 1):
                    sq = s0_f if q == 0 else s1_f
                    lax.cond(sq == 1, lambda _q=q: wait_scatter(_q), lambda: None)

                def final_flush(_tab=tab):
                    pltpu.sync_copy(_tab.at[csrcF], rows0)
                    pltpu.sync_copy(rows0, acc.at[cdstF], add=True)
                    if cnt_off is not None:
                        pltpu.sync_copy(ones1, acc1d.at[cdstF], add=True)
                    lax.fori_loop(0, K // 16, reset_fill, 0)
                lax.cond(cnt_f > 0, final_flush, lambda: None)

            plsc.subcore_barrier()
            obase = base + c * R + s * r16
            pltpu.sync_copy(acc.at[pl.ds(s * r16, r16)],
                            souts[sum_slot].at[pl.ds(obase, r16)])
            plsc.subcore_barrier()

        for op in program:
            if op[0] == "phase":
                _, sum_slot, base, R, rel_ids, cnt_off = op
                emit_phase(sum_slot, base, R, rel_ids, cnt_off)
            elif op[0] == "zero1d":
                _, L = op
                t16 = L // NS
                nfull, rem = t16 // 512, t16 % 512
                if nfull:
                    def zero_cnt(j, _, _t16=t16):
                        pltpu.sync_copy(z1d, acc1d.at[pl.ds(s * _t16 + j * 512, 512)])
                        return 0
                    lax.fori_loop(0, nfull, zero_cnt, 0)
                if rem:
                    pltpu.sync_copy(z1d.at[pl.ds(0, rem)],
                                    acc1d.at[pl.ds(s * t16 + nfull * 512, rem)])
            elif op[0] == "wb1d":
                _, cnt_slot, src_off, L, dst_mul = op
                t16 = L // NS
                dbase = dst_mul + c * L + s * t16
                pltpu.sync_copy(acc1d.at[pl.ds(src_off + s * t16, t16)],
                                cbuf.at[pl.ds(0, t16)])
                pltpu.sync_copy(cbuf.at[pl.ds(0, t16)],
                                couts[cnt_slot].at[pl.ds(dbase, t16)])
                plsc.subcore_barrier()

    out_type = [jax.ShapeDtypeStruct((r, HID), jnp.float32) for r in sum_rows]
    out_type += [jax.ShapeDtypeStruct((r,), jnp.float32) for r in cnt_rows]
    scratch = [pltpu.VMEM_SHARED((acc_rows, HID), jnp.float32)]
    if with_counts:
        scratch.append(pltpu.VMEM_SHARED((acc1d_rows,), jnp.float32))
    scratch += [
        pltpu.VMEM((2 * EB,), jnp.int32),
        pltpu.VMEM((K,), jnp.int32),
        pltpu.VMEM((K,), jnp.int32),
        pltpu.VMEM((K,), jnp.int32),
        pltpu.VMEM((K,), jnp.int32),
        pltpu.VMEM((K, HID), jnp.float32),
        pltpu.VMEM((K,), jnp.int32),
        pltpu.VMEM((K,), jnp.int32),
        pltpu.VMEM((K, HID), jnp.float32),
    ]
    if with_counts:
        scratch.append(pltpu.VMEM((K,), jnp.float32))
    scratch.append(pltpu.VMEM((8, HID), jnp.float32))
    if with_counts:
        scratch.append(pltpu.VMEM((512,), jnp.float32))
        scratch.append(pltpu.VMEM((800,), jnp.float32))
    scratch += [pltpu.SemaphoreType.DMA] * 4

    return pl.kernel(body, out_type=out_type, mesh=_mesh(), scratch_types=scratch,
                     compiler_params=_params())


def _sc_batch_gather(h_d2, h_c2, drug1, drug2, cell):
    per = BATCH // (NC * NS)

    def body(hd, hc, i1, i2, ic, o1, o2, oc, idx_v, rows_v):
        c = lax.axis_index("c")
        s = lax.axis_index("s")
        wid = s * NC + c
        base = wid * per
        for (ib, tab, ob) in ((i1, hd, o1), (i2, hd, o2), (ic, hc, oc)):
            pltpu.sync_copy(ib.at[pl.ds(base, per)], idx_v)
            pltpu.sync_copy(tab.at[idx_v], rows_v)
            pltpu.sync_copy(rows_v, ob.at[pl.ds(base, per)])

    out_type = [jax.ShapeDtypeStruct((BATCH, HID), jnp.float32)] * 3
    scratch = [pltpu.VMEM((per,), jnp.int32), pltpu.VMEM((per, HID), jnp.float32)]
    return pl.kernel(body, out_type=out_type, mesh=_mesh(), scratch_types=scratch,
                     compiler_params=_params())(h_d2, h_c2, drug1, drug2, cell)


def _layer_update_body(h_ref, s_ref, cnt_ref, w_ref, o_ref):
    h = h_ref[...]
    sm = s_ref[...]
    deg = jnp.maximum(cnt_ref[...], 1.0)
    x = h + sm / deg
    o_ref[...] = jnp.maximum(jnp.dot(x, w_ref[...], preferred_element_type=jnp.float32), 0.0)


def _layer_update(h, ssum, cnt, W, block=512):
    n = h.shape[0]
    grid = (pl.cdiv(n, block),)
    return pl.pallas_call(
        _layer_update_body,
        grid=grid,
        in_specs=[
            pl.BlockSpec((block, HID), lambda i: (i, 0)),
            pl.BlockSpec((block, HID), lambda i: (i, 0)),
            pl.BlockSpec((block, 1), lambda i: (i, 0)),
            pl.BlockSpec((HID, HID), lambda i: (0, 0)),
        ],
        out_specs=pl.BlockSpec((block, HID), lambda i: (i, 0)),
        out_shape=jax.ShapeDtypeStruct((n, HID), jnp.float32),
    )(h, ssum, cnt, W)


def _mlp_body(u1_ref, u2_ref, uc_ref, w1_ref, b1_ref, w2_ref, b2_ref, w3_ref, b3_ref, o_ref):
    def l2n(x):
        nrm = jnp.sqrt(jnp.sum(x * x, axis=1, keepdims=True))
        return x / jnp.maximum(nrm, 1e-12)

    hid = jnp.concatenate([l2n(u1_ref[...]), l2n(u2_ref[...]), l2n(uc_ref[...])], axis=1)
    h = jnp.maximum(jnp.dot(hid, w1_ref[...], preferred_element_type=jnp.float32) + b1_ref[...], 0.0)
    h = jnp.maximum(jnp.dot(h, w2_ref[...], preferred_element_type=jnp.float32) + b2_ref[...], 0.0)
    o_ref[...] = jnp.dot(h, w3_ref[...], preferred_element_type=jnp.float32) + b3_ref[...]


def _mlp(u1, u2, uc, w1, b1, w2, b2, w3, b3, block=512):
    grid = (BATCH // block,)
    return pl.pallas_call(
        _mlp_body,
        grid=grid,
        in_specs=[
            pl.BlockSpec((block, HID), lambda i: (i, 0)),
            pl.BlockSpec((block, HID), lambda i: (i, 0)),
            pl.BlockSpec((block, HID), lambda i: (i, 0)),
            pl.BlockSpec(w1.shape, lambda i: (0, 0)),
            pl.BlockSpec(b1.shape, lambda i: (0,)),
            pl.BlockSpec(w2.shape, lambda i: (0, 0)),
            pl.BlockSpec(b2.shape, lambda i: (0,)),
            pl.BlockSpec(w3.shape, lambda i: (0, 0)),
            pl.BlockSpec(b3.shape, lambda i: (0,)),
        ],
        out_specs=pl.BlockSpec((block, 2), lambda i: (i, 0)),
        out_shape=jax.ShapeDtypeStruct((BATCH, 2), jnp.float32),
    )(u1, u2, uc, w1, b1, w2, b2, w3, b3)


def kernel(drug_table, protein_table, cell_table, gnn_w, w1, b1, w2, b2, w3, b3,
           x_drug, x_protein, x_cell, edge_index_dp, edge_index_pd, edge_index_pp,
           edge_index_cp, edge_index_pc, drug1, drug2, cell):
    n_d, n_p, n_c = drug_table.shape[0], protein_table.shape[0], cell_table.shape[0]
    h_d, h_p, h_c = drug_table, protein_table, cell_table

    epw_dp = epw_pd = 10240
    epw_pp = 12800
    epw_cp = epw_pc = 3840
    dp_e = _pack_edges(edge_index_dp, epw_dp)
    pd_e = _pack_edges(edge_index_pd, epw_pd)
    pp_e = _pack_edges(edge_index_pp, epw_pp)
    cp_e = _pack_edges(edge_index_cp, epw_cp)
    pc_e = _pack_edges(edge_index_pc, epw_pc)

    z128 = jnp.zeros((8, HID), jnp.float32)

    seg1 = _make_sc_seg_sum(
        table_shapes=[(n_d, HID), (n_p, HID), (n_c, HID)],
        rels=[(1, epw_pd), (0, epw_dp), (1, epw_pp), (2, epw_cp), (1, epw_pc)],
        program=[
            ("zero1d", 12544),
            ("phase", 1, 0, 12544, [1, 2, 3], 0),
            ("wb1d", 1, 0, 12544, 0),
            ("zero1d", 12544),
            ("phase", 1, 25088, 12544, [1, 2, 3], 0),
            ("wb1d", 1, 0, 12544, 25088),
            ("zero1d", 5120),
            ("phase", 0, 0, 5120, [0], 0),
            ("wb1d", 0, 0, 5120, 0),
            ("zero1d", 512),
            ("phase", 2, 0, 512, [4], 0),
            ("wb1d", 2, 0, 512, 0),
        ],
        sum_rows=[10240, 50176, 1024],
        cnt_rows=[10240, 50176, 1024],
        acc_rows=12552,
        acc1d_rows=12552,
    )
    s_d, s_p, s_c, c_d, c_p, c_c = seg1(
        h_d, h_p, h_c, pd_e, dp_e, pp_e, cp_e, pc_e, z128)
    c_d2d, c_p2d, c_c2d = c_d[:, None], c_p[:, None], c_c[:, None]

    h_d1 = _layer_update(h_d, s_d, c_d2d, gnn_w[0])
    h_p1 = _layer_update(h_p, s_p, c_p2d, gnn_w[0])
    h_c1 = _layer_update(h_c, s_c, c_c2d, gnn_w[0])

    seg2 = _make_sc_seg_sum(
        table_shapes=[(n_p, HID)],
        rels=[(0, epw_pd), (0, epw_pc)],
        program=[
            ("phase", 0, 0, 5120, [0], None),
            ("phase", 1, 0, 512, [1], None),
        ],
        sum_rows=[10240, 1024],
        cnt_rows=[],
        acc_rows=5128,
        acc1d_rows=16,
    )
    s_d2, s_c2 = seg2(h_p1, pd_e, pc_e, z128)

    h_d2 = _layer_update(h_d1, s_d2, c_d2d, gnn_w[1])
    h_c2 = _layer_update(h_c1, s_c2, c_c2d, gnn_w[1])

    u1, u2, uc = _sc_batch_gather(h_d2, h_c2, drug1, drug2, cell)
    return _mlp(u1, u2, uc, w1, b1, w2, b2, w3, b3)
